# baseline (speedup 1.0000x reference)
import functools

import jax
import jax.numpy as jnp
from jax import lax
from jax.experimental import pallas as pl
from jax.experimental.pallas import tpu as pltpu
from jax.experimental.pallas import tpu_sc as plsc


@functools.lru_cache(maxsize=None)
def _build_gather(B, V, D):
    NC, NS = 2, 16
    NW = NC * NS
    b_per_w = B // NW
    CH = 128
    n_ch = b_per_w // CH
    NBUF = 2

    mesh = plsc.VectorSubcoreMesh(core_axis_name="c", subcore_axis_name="s")

    @functools.partial(
        pl.kernel,
        mesh=mesh,
        out_type=jax.ShapeDtypeStruct((B, D), jnp.float32),
        scratch_types=[
            pltpu.VMEM((b_per_w,), jnp.int32),
            pltpu.VMEM((NBUF, CH, D), jnp.float32),
        ]
        + [pltpu.SemaphoreType.DMA] * (2 * NBUF),
    )
    def k(mem, idx_hbm, rows_hbm, idx_v, rows_v, *sems):
        g = sems[0:NBUF]
        o = sems[NBUF:2 * NBUF]

        wid = lax.axis_index("s") * NC + lax.axis_index("c")
        base = wid * b_per_w
        pltpu.sync_copy(idx_hbm.at[pl.ds(base, b_per_w)], idx_v)

        handles = {}

        def issue_gather(c):
            buf = c % NBUF
            handles[("g", c)] = pltpu.async_copy(
                mem.at[idx_v.at[pl.ds(c * CH, CH)]], rows_v.at[buf], g[buf])

        issue_gather(0)
        for c in range(n_ch):
            buf = c % NBUF
            handles[("g", c)].wait()
            if c + 1 < n_ch:
                if c >= 1:
                    handles[("o", c - 1)].wait()
                issue_gather(c + 1)
            handles[("o", c)] = pltpu.async_copy(
                rows_v.at[buf], rows_hbm.at[pl.ds(base + c * CH, CH)], o[buf])

        handles[("o", n_ch - 2)].wait()
        handles[("o", n_ch - 1)].wait()

    return k


@functools.lru_cache(maxsize=None)
def _build_epilogue(B, D, half):
    BLK = 512

    def body(rows_ref, emb_ref, num_ref):
        x = rows_ref[...]
        eye = (lax.broadcasted_iota(jnp.int32, (half, half), 0)
               == lax.broadcasted_iota(jnp.int32, (half, half), 1)
               ).astype(jnp.float32)
        dn = (((1,), (1,)), ((), ()))
        num_t = lax.dot_general(eye, x[:, :half], dn,
                                preferred_element_type=jnp.float32)
        den_t = lax.dot_general(eye, x[:, half:], dn,
                                preferred_element_type=jnp.float32)
        num_ref[...] = num_t
        emb_ref[...] = num_t / den_t

    return pl.pallas_call(
        body,
        grid=(B // BLK,),
        in_specs=[pl.BlockSpec((BLK, D), lambda i: (i, 0))],
        out_specs=[
            pl.BlockSpec((half, BLK), lambda i: (0, i)),
            pl.BlockSpec((half, BLK), lambda i: (0, i)),
        ],
        out_shape=(
            jax.ShapeDtypeStruct((half, B), jnp.float32),
            jax.ShapeDtypeStruct((half, B), jnp.float32),
        ),
    )


def kernel(memory, nodes, memory_dim):
    V, D = memory.shape
    B = nodes.shape[0]
    half = D // 2
    rows = _build_gather(B, V, D)(memory, nodes.astype(jnp.int32))
    emb_t, num_t = _build_epilogue(B, D, half)(rows)
    return (emb_t.T, num_t.T)


# speedup vs baseline: 1.1242x; 1.1242x over previous
import functools

import jax
import jax.numpy as jnp
from jax import lax
from jax.experimental import pallas as pl
from jax.experimental.pallas import tpu as pltpu
from jax.experimental.pallas import tpu_sc as plsc

_L = 16


@functools.lru_cache(maxsize=None)
def _build(B, V, D, half):
    NC, NS = 2, 16
    NW = NC * NS
    b_per_w = B // NW
    CH = 128
    n_ch = b_per_w // CH
    NBUF = 2
    UNROLL = 4

    mesh = plsc.VectorSubcoreMesh(core_axis_name="c", subcore_axis_name="s")

    @functools.partial(
        pl.kernel,
        mesh=mesh,
        out_type=(
            jax.ShapeDtypeStruct((B, half), jnp.float32),
            jax.ShapeDtypeStruct((B, half), jnp.float32),
        ),
        scratch_types=[
            pltpu.VMEM((b_per_w,), jnp.int32),
            pltpu.VMEM((NBUF, CH, D), jnp.float32),
            pltpu.VMEM((NBUF, CH, half), jnp.float32),
            pltpu.VMEM((NBUF, CH, half), jnp.float32),
        ]
        + [pltpu.SemaphoreType.DMA] * (3 * NBUF),
    )
    def k(mem, idx_hbm, emb_hbm, num_hbm, idx_v, rows_v, emb_v, num_v, *sems):
        g = sems[0:NBUF]
        on = sems[NBUF:2 * NBUF]
        oe = sems[2 * NBUF:3 * NBUF]

        wid = lax.axis_index("s") * NC + lax.axis_index("c")
        base = wid * b_per_w
        pltpu.sync_copy(idx_hbm.at[pl.ds(base, b_per_w)], idx_v)

        handles = {}

        def issue_gather(c):
            buf = c % NBUF
            handles[("g", c)] = pltpu.async_copy(
                mem.at[idx_v.at[pl.ds(c * CH, CH)]], rows_v.at[buf], g[buf])

        issue_gather(0)
        for c in range(n_ch):
            buf = c % NBUF
            cb = c * CH
            handles[("g", c)].wait()
            if c + 1 < n_ch:
                issue_gather(c + 1)
            if c >= NBUF:
                handles[("on", c - NBUF)].wait()
                handles[("oe", c - NBUF)].wait()

            def body(i, _):
                for r in range(UNROLL):
                    row = UNROLL * i + r
                    for j in range(half // _L):
                        s = pl.ds(j * _L, _L)
                        num = rows_v[buf, row, s]
                        den = rows_v[buf, row, pl.ds(half + j * _L, _L)]
                        num_v[buf, row, s] = num
                        emb_v[buf, row, s] = num / den
                return 0

            lax.fori_loop(0, CH // UNROLL, body, 0)
            handles[("on", c)] = pltpu.async_copy(
                num_v.at[buf], num_hbm.at[pl.ds(base + cb, CH)], on[buf])
            handles[("oe", c)] = pltpu.async_copy(
                emb_v.at[buf], emb_hbm.at[pl.ds(base + cb, CH)], oe[buf])

        for c in range(n_ch - NBUF, n_ch):
            handles[("on", c)].wait()
            handles[("oe", c)].wait()

    return k


def kernel(memory, nodes, memory_dim):
    V, D = memory.shape
    B = nodes.shape[0]
    half = D // 2
    k = _build(B, V, D, half)
    emb, num = k(memory, nodes.astype(jnp.int32))
    return (emb, num)


# speedup vs baseline: 1.3451x; 1.1966x over previous
import functools

import jax
import jax.numpy as jnp
from jax import lax
from jax.experimental import pallas as pl
from jax.experimental.pallas import tpu as pltpu
from jax.experimental.pallas import tpu_sc as plsc


@functools.lru_cache(maxsize=None)
def _build_gather(B, V, D):
    NC, NS = 2, 16
    NW = NC * NS
    b_per_w = B // NW
    CH = 64
    n_ch = b_per_w // CH
    NBUF = 4

    mesh = plsc.VectorSubcoreMesh(core_axis_name="c", subcore_axis_name="s")

    @functools.partial(
        pl.kernel,
        mesh=mesh,
        out_type=jax.ShapeDtypeStruct((B, D), jnp.float32),
        scratch_types=[
            pltpu.VMEM((b_per_w,), jnp.int32),
            pltpu.VMEM((NBUF, CH, D), jnp.float32),
        ]
        + [pltpu.SemaphoreType.DMA] * (2 * NBUF),
    )
    def k(mem, idx_hbm, rows_hbm, idx_v, rows_v, *sems):
        g = sems[0:NBUF]
        o = sems[NBUF:2 * NBUF]

        wid = lax.axis_index("s") * NC + lax.axis_index("c")
        base = wid * b_per_w
        pltpu.sync_copy(idx_hbm.at[pl.ds(base, b_per_w)], idx_v)

        handles = {}

        def issue_gather(c):
            buf = c % NBUF
            handles[("g", c)] = pltpu.async_copy(
                mem.at[idx_v.at[pl.ds(c * CH, CH)]], rows_v.at[buf], g[buf])

        for c in range(min(NBUF, n_ch)):
            issue_gather(c)
        for c in range(n_ch):
            buf = c % NBUF
            handles[("g", c)].wait()
            handles[("o", c)] = pltpu.async_copy(
                rows_v.at[buf], rows_hbm.at[pl.ds(base + c * CH, CH)], o[buf])
            if c + NBUF < n_ch:
                handles[("o", c)].wait()
                issue_gather(c + NBUF)
        for c in range(max(0, n_ch - NBUF), n_ch):
            handles[("o", c)].wait()

    return k


@functools.lru_cache(maxsize=None)
def _build_epilogue(B, D, half):
    BLK = 2048

    def body(rows_ref, emb_ref, num_ref):
        x = rows_ref[...]
        eye = (lax.broadcasted_iota(jnp.int32, (half, half), 0)
               == lax.broadcasted_iota(jnp.int32, (half, half), 1)
               ).astype(jnp.float32)
        dn = (((1,), (1,)), ((), ()))
        num_t = lax.dot_general(eye, x[:, :half], dn,
                                preferred_element_type=jnp.float32)
        den_t = lax.dot_general(eye, x[:, half:], dn,
                                preferred_element_type=jnp.float32)
        num_ref[...] = num_t
        emb_ref[...] = num_t / den_t

    return pl.pallas_call(
        body,
        grid=(B // BLK,),
        in_specs=[pl.BlockSpec((BLK, D), lambda i: (i, 0))],
        out_specs=[
            pl.BlockSpec((half, BLK), lambda i: (0, i)),
            pl.BlockSpec((half, BLK), lambda i: (0, i)),
        ],
        out_shape=(
            jax.ShapeDtypeStruct((half, B), jnp.float32),
            jax.ShapeDtypeStruct((half, B), jnp.float32),
        ),
    )


def kernel(memory, nodes, memory_dim):
    V, D = memory.shape
    B = nodes.shape[0]
    half = D // 2
    rows = _build_gather(B, V, D)(memory, nodes.astype(jnp.int32))
    emb_t, num_t = _build_epilogue(B, D, half)(rows)
    return (emb_t.T, num_t.T)
